# baseline (speedup 1.0000x reference)
import functools

import jax
import jax.numpy as jnp
from jax import lax
from jax.experimental import pallas as pl
from jax.experimental.pallas import tpu as pltpu
from jax.experimental.pallas import tpu_sc as plsc

HIDDEN = 128
LANES = 16
NUM_CORES = 2
NUM_SUBCORES = 16
NUM_WORKERS = NUM_CORES * NUM_SUBCORES
CHUNK = 80


def _combine_tables(level_emb, cate_emb):
    nl, nc = level_emb.shape[0], cate_emb.shape[0]

    def body(lvl_ref, cat_ref, out_ref):
        out_ref[...] = lvl_ref[...][:, None, :] + cat_ref[...][None, :, :]

    comb3 = pl.pallas_call(
        body,
        out_shape=jax.ShapeDtypeStruct((nl, nc, HIDDEN), jnp.float32),
    )(level_emb, cate_emb)
    return comb3.reshape(nl * nc, HIDDEN)


def kernel(x, cate_emb, level_emb, pe):
    n = x.shape[0]
    nl, nc, npe = level_emb.shape[0], cate_emb.shape[0], pe.shape[0]

    xi = x.astype(jnp.int32)
    fidx = jnp.clip(xi[:, 0], 0, nl - 1) * nc + jnp.clip(xi[:, 1], 0, nc - 1)
    tho = jnp.clip(xi[:, 2], 0, npe - 1)

    comb = _combine_tables(level_emb, cate_emb)

    k_per_w = -(-n // (NUM_WORKERS * CHUNK))
    total = NUM_WORKERS * k_per_w * CHUNK
    fidx_p = jnp.pad(fidx, (0, total - n))
    tho_p = jnp.pad(tho, (0, total - n))
    idx_packed = jnp.stack(
        [fidx_p.reshape(-1, CHUNK), tho_p.reshape(-1, CHUNK)], axis=1
    )

    mesh = plsc.VectorSubcoreMesh(core_axis_name="c", subcore_axis_name="s")

    @functools.partial(
        pl.kernel,
        out_type=(
            jax.ShapeDtypeStruct((n, HIDDEN), jnp.float32),
            jax.ShapeDtypeStruct((CHUNK, HIDDEN), jnp.float32),
        ),
        mesh=mesh,
        compiler_params=pltpu.CompilerParams(use_tc_tiling_on_sc=False),
        scratch_types=[
            pltpu.VMEM((2, CHUNK), jnp.int32),
            pltpu.VMEM((CHUNK, HIDDEN), jnp.float32),
            pltpu.VMEM((CHUNK, HIDDEN), jnp.float32),
            pltpu.VMEM((CHUNK, HIDDEN), jnp.float32),
        ],
    )
    def enc(idx_hbm, comb_hbm, pe_hbm, out_hbm, dum_hbm,
            idxbuf, g_a, g_b, st):
        w = lax.axis_index("s") * NUM_CORES + lax.axis_index("c")

        @pl.loop(0, k_per_w)
        def _chunk(k):
            pltpu.sync_copy(idx_hbm.at[w * k_per_w + k], idxbuf)
            pltpu.sync_copy(comb_hbm.at[idxbuf.at[0]], g_a)
            pltpu.sync_copy(pe_hbm.at[idxbuf.at[1]], g_b)

            @pl.loop(0, CHUNK)
            def _row(r):
                for h in range(HIDDEN // LANES):
                    sl = (r, pl.ds(h * LANES, LANES))
                    st[sl] = g_a[sl] + g_b[sl]

            base = (w * k_per_w + k) * CHUNK
            is_real = base < n

            @pl.when(is_real)
            def _():
                pltpu.sync_copy(st, out_hbm.at[pl.ds(base, CHUNK)])

            @pl.when(jnp.logical_not(is_real))
            def _():
                pltpu.sync_copy(st, dum_hbm)

    out, _ = enc(idx_packed, comb, pe)
    return out


# speedup vs baseline: 20.7032x; 20.7032x over previous
import functools

import jax
import jax.numpy as jnp
from jax import lax
from jax.experimental import pallas as pl
from jax.experimental.pallas import tpu as pltpu
from jax.experimental.pallas import tpu_sc as plsc

HIDDEN = 128
LANES = 16
NUM_CORES = 2
NUM_SUBCORES = 16
NUM_WORKERS = NUM_CORES * NUM_SUBCORES
CHUNK = 80


def _combine_tables(level_emb, cate_emb):
    nl, nc = level_emb.shape[0], cate_emb.shape[0]

    def body(lvl_ref, cat_ref, out_ref):
        out_ref[...] = lvl_ref[...][:, None, :] + cat_ref[...][None, :, :]

    comb3 = pl.pallas_call(
        body,
        out_shape=jax.ShapeDtypeStruct((nl, nc, HIDDEN), jnp.float32),
    )(level_emb, cate_emb)
    return comb3.reshape(nl * nc, HIDDEN)


def kernel(x, cate_emb, level_emb, pe):
    n = x.shape[0]
    nl, nc, npe = level_emb.shape[0], cate_emb.shape[0], pe.shape[0]

    xi = x.astype(jnp.int32)
    fidx = jnp.clip(xi[:, 0], 0, nl - 1) * nc + jnp.clip(xi[:, 1], 0, nc - 1)
    tho = jnp.clip(xi[:, 2], 0, npe - 1)

    comb = _combine_tables(level_emb, cate_emb)

    k_per_w = -(-n // (NUM_WORKERS * CHUNK))
    total = NUM_WORKERS * k_per_w * CHUNK
    fidx_p = jnp.pad(fidx, (0, total - n))
    tho_p = jnp.pad(tho, (0, total - n))
    idx_packed = jnp.stack(
        [fidx_p.reshape(-1, CHUNK), tho_p.reshape(-1, CHUNK)], axis=1
    )

    mesh = plsc.VectorSubcoreMesh(core_axis_name="c", subcore_axis_name="s")

    @functools.partial(
        pl.kernel,
        out_type=(
            jax.ShapeDtypeStruct((n, HIDDEN), jnp.float32),
            jax.ShapeDtypeStruct((CHUNK, HIDDEN), jnp.float32),
        ),
        mesh=mesh,
        compiler_params=pltpu.CompilerParams(use_tc_tiling_on_sc=False),
        scratch_types=[
            pltpu.VMEM((2, CHUNK), jnp.int32),
            pltpu.VMEM((CHUNK, HIDDEN), jnp.float32),
            pltpu.VMEM((CHUNK, HIDDEN), jnp.float32),
            pltpu.VMEM((CHUNK, HIDDEN), jnp.float32),
            pltpu.VMEM_SHARED((nl * nc, HIDDEN), jnp.float32),
            pltpu.VMEM_SHARED((npe, HIDDEN), jnp.float32),
        ],
    )
    def enc(idx_hbm, comb_hbm, pe_hbm, out_hbm, dum_hbm,
            idxbuf, g_a, g_b, st, comb_sp, pe_sp):
        sid = lax.axis_index("s")
        w = sid * NUM_CORES + lax.axis_index("c")

        @pl.when(sid == 0)
        def _():
            pltpu.sync_copy(comb_hbm, comb_sp)
            pltpu.sync_copy(pe_hbm, pe_sp)

        plsc.subcore_barrier()

        @pl.loop(0, k_per_w)
        def _chunk(k):
            pltpu.sync_copy(idx_hbm.at[w * k_per_w + k], idxbuf)
            pltpu.sync_copy(comb_sp.at[idxbuf.at[0]], g_a)
            pltpu.sync_copy(pe_sp.at[idxbuf.at[1]], g_b)

            @pl.loop(0, CHUNK)
            def _row(r):
                for h in range(HIDDEN // LANES):
                    sl = (r, pl.ds(h * LANES, LANES))
                    st[sl] = g_a[sl] + g_b[sl]

            base = (w * k_per_w + k) * CHUNK
            is_real = base < n

            @pl.when(is_real)
            def _():
                pltpu.sync_copy(st, out_hbm.at[pl.ds(base, CHUNK)])

            @pl.when(jnp.logical_not(is_real))
            def _():
                pltpu.sync_copy(st, dum_hbm)

    out, _ = enc(idx_packed, comb, pe)
    return out


# speedup vs baseline: 25.9117x; 1.2516x over previous
import functools

import jax
import jax.numpy as jnp
from jax import lax
from jax.experimental import pallas as pl
from jax.experimental.pallas import tpu as pltpu
from jax.experimental.pallas import tpu_sc as plsc

HIDDEN = 128
LANES = 16
NUM_CORES = 2
NUM_SUBCORES = 16
NUM_WORKERS = NUM_CORES * NUM_SUBCORES
CHUNK = 80
NBUF = 2


def _combine_tables(level_emb, cate_emb):
    nl, nc = level_emb.shape[0], cate_emb.shape[0]

    def body(lvl_ref, cat_ref, out_ref):
        out_ref[...] = lvl_ref[...][:, None, :] + cat_ref[...][None, :, :]

    comb3 = pl.pallas_call(
        body,
        out_shape=jax.ShapeDtypeStruct((nl, nc, HIDDEN), jnp.float32),
    )(level_emb, cate_emb)
    return comb3.reshape(nl * nc, HIDDEN)


def kernel(x, cate_emb, level_emb, pe):
    n = x.shape[0]
    nl, nc, npe = level_emb.shape[0], cate_emb.shape[0], pe.shape[0]
    ncomb = nl * nc

    xi = x.astype(jnp.int32)
    fidx = jnp.clip(xi[:, 0], 0, nl - 1) * nc + jnp.clip(xi[:, 1], 0, nc - 1)
    tho = jnp.clip(xi[:, 2], 0, npe - 1)

    comb = _combine_tables(level_emb, cate_emb)

    k_per_w = -(-n // (NUM_WORKERS * CHUNK))
    total = NUM_WORKERS * k_per_w * CHUNK
    fidx_p = jnp.pad(fidx, (0, total - n))
    tho_p = jnp.pad(tho, (0, total - n))
    idx_packed = jnp.stack(
        [fidx_p.reshape(-1, CHUNK), tho_p.reshape(-1, CHUNK)], axis=1
    )

    comb_part = ncomb // 8
    pe_part = npe // 8

    mesh = plsc.VectorSubcoreMesh(core_axis_name="c", subcore_axis_name="s")

    @functools.partial(
        pl.kernel,
        out_type=(
            jax.ShapeDtypeStruct((n, HIDDEN), jnp.float32),
            jax.ShapeDtypeStruct((CHUNK, HIDDEN), jnp.float32),
        ),
        mesh=mesh,
        compiler_params=pltpu.CompilerParams(use_tc_tiling_on_sc=False),
        scratch_types=[
            pltpu.VMEM((k_per_w, 2, CHUNK), jnp.int32),
            pltpu.VMEM((CHUNK, HIDDEN), jnp.float32),
            pltpu.VMEM((CHUNK, HIDDEN), jnp.float32),
            pltpu.VMEM((NBUF, CHUNK, HIDDEN), jnp.float32),
            pltpu.VMEM_SHARED((ncomb, HIDDEN), jnp.float32),
            pltpu.VMEM_SHARED((npe, HIDDEN), jnp.float32),
            pltpu.SemaphoreType.DMA,
            pltpu.SemaphoreType.DMA,
        ],
    )
    def enc(idx_hbm, comb_hbm, pe_hbm, out_hbm, dum_hbm,
            idxbuf, g_a, g_b, st, comb_sp, pe_sp, sem_o0, sem_o1):
        sem_o = (sem_o0, sem_o1)
        sid = lax.axis_index("s")
        w = sid * NUM_CORES + lax.axis_index("c")

        pltpu.sync_copy(idx_hbm.at[pl.ds(w * k_per_w, k_per_w)], idxbuf)

        @pl.when(sid < 8)
        def _():
            pltpu.sync_copy(
                comb_hbm.at[pl.ds(sid * comb_part, comb_part)],
                comb_sp.at[pl.ds(sid * comb_part, comb_part)],
            )

        @pl.when(sid >= 8)
        def _():
            pltpu.sync_copy(
                pe_hbm.at[pl.ds((sid - 8) * pe_part, pe_part)],
                pe_sp.at[pl.ds((sid - 8) * pe_part, pe_part)],
            )

        plsc.subcore_barrier()

        def out_issue(b, k):
            base = (w * k_per_w + k) * CHUNK
            is_real = base < n

            @pl.when(is_real)
            def _():
                pltpu.async_copy(st.at[b], out_hbm.at[pl.ds(base, CHUNK)],
                                 sem_o[b])

            @pl.when(jnp.logical_not(is_real))
            def _():
                pltpu.async_copy(st.at[b], dum_hbm, sem_o[b])

        @pl.loop(0, k_per_w, step=NBUF)
        def _pair(t):
            for b in range(NBUF):
                k = t + b
                pltpu.sync_copy(comb_sp.at[idxbuf.at[k, 0]], g_a)
                pltpu.sync_copy(pe_sp.at[idxbuf.at[k, 1]], g_b)

                @pl.when(t > 0)
                def _():
                    pltpu.make_async_copy(st.at[b], dum_hbm, sem_o[b]).wait()

                @pl.loop(0, CHUNK)
                def _row(r):
                    for h in range(HIDDEN // LANES):
                        sl = (r, pl.ds(h * LANES, LANES))
                        st[(b, *sl)] = g_a[sl] + g_b[sl]

                out_issue(b, k)

        pltpu.make_async_copy(st.at[0], dum_hbm, sem_o[0]).wait()
        pltpu.make_async_copy(st.at[1], dum_hbm, sem_o[1]).wait()

    out, _ = enc(idx_packed, comb, pe)
    return out
